# baseline (speedup 1.0000x reference)
import dataclasses

import jax
import jax.numpy as jnp
from jax.experimental import pallas as pl
from jax.experimental.pallas import tpu as pltpu
from jax.experimental.pallas import tpu_sc as plsc

_M, _NC, _K = 1024, 512, 32
_HI = jax.lax.Precision.HIGHEST


def _rank_desc_row(y_col, y_row, n):
    gt = (y_col > y_row).astype(jnp.float32)
    k_idx = jax.lax.broadcasted_iota(jnp.int32, (n, n), 0)
    j_idx = jax.lax.broadcasted_iota(jnp.int32, (n, n), 1)
    tie = ((y_col == y_row) & (k_idx < j_idx)).astype(jnp.float32)
    return jnp.sum(gt + tie, axis=0, keepdims=True).astype(jnp.int32)


def _main_body(theta_t_ref, seed_ref, ic_ref, fc_ref,
               t12_ref, dxv_ref, dyv_ref, sxv_ref, syv_ref,
               rdx_ref, rdy_ref, rsx_ref, rsy_ref):
    theta = jnp.log(jax.nn.sigmoid(theta_t_ref[...]))
    t0 = theta[0:1, :]
    t1 = theta[1:2, :]
    t2 = theta[2:3, :]
    c0 = jnp.sum(t0)
    u = t1 - t0
    v = t0 - 2.0 * t1 + t2

    fc = fc_ref[...]
    dimn = (((1,), (1,)), ((), ()))
    s_col = jax.lax.dot_general(fc, u, dimn, precision=_HI)
    s_row = jax.lax.dot_general(u, fc, dimn, precision=_HI)
    g = jax.lax.dot_general(fc * v, fc, dimn, precision=_HI)
    p = c0 + s_col + s_row + g
    i_idx = jax.lax.broadcasted_iota(jnp.int32, (_NC, _NC), 0)
    j_idx = jax.lax.broadcasted_iota(jnp.int32, (_NC, _NC), 1)
    p = jnp.where(i_idx == j_idx, 0.0, p)
    e = jnp.exp(p)

    sp = seed_ref[...]
    sp_max = jnp.max(sp)
    es = jnp.exp(sp - sp_max)
    seed_row = es / jnp.sum(es)
    mask = (ic_ref[...] == 1).astype(jnp.float32)
    group_sum = jax.lax.dot_general(mask, seed_row, dimn, precision=_HI)
    w = mask * (seed_row / group_sum)

    s_mat = jax.lax.dot_general(w, e, (((1,), (0,)), ((), ())),
                                precision=_HI)

    log_s = jnp.log(jnp.where(mask > 0, s_mat, 1.0))
    others = jnp.log1p(-jnp.where(mask > 0, 0.0, s_mat))
    loss = -jnp.sum(log_s) - jnp.sum(others)

    ones_m = jnp.ones((1, _M), dtype=jnp.float32)
    ones_nc = jnp.ones((1, _NC), dtype=jnp.float32)
    dim_c0 = (((1,), (0,)), ((), ()))
    dim_rev = (((0,), (1,)), ((), ()))
    d_x_row = jax.lax.dot_general(ones_m, s_mat, dim_c0, precision=_HI)
    d_x_col = jax.lax.dot_general(s_mat, ones_m, dim_rev, precision=_HI)
    d_y_row = jax.lax.dot_general(ones_m, mask, dim_c0, precision=_HI)
    d_y_col = jax.lax.dot_general(mask, ones_m, dim_rev, precision=_HI)
    s_x_col = jax.lax.dot_general(s_mat, ones_nc, dimn, precision=_HI)
    s_x_row = jax.lax.dot_general(ones_nc, s_mat, dimn, precision=_HI)
    s_y_col = jax.lax.dot_general(mask, ones_nc, dimn, precision=_HI)
    s_y_row = jax.lax.dot_general(ones_nc, mask, dimn, precision=_HI)

    t12_ref[...] = jnp.full((1, 16), loss, dtype=jnp.float32)
    dxv_ref[...] = d_x_row
    dyv_ref[...] = d_y_row
    sxv_ref[...] = s_x_row
    syv_ref[...] = s_y_row
    rdx_ref[...] = _rank_desc_row(d_x_col, d_x_row, _NC)
    rdy_ref[...] = _rank_desc_row(d_y_col, d_y_row, _NC)
    rsx_ref[...] = _rank_desc_row(s_x_col, s_x_row, _M)
    rsy_ref[...] = _rank_desc_row(s_y_col, s_y_row, _M)


def _sc_sort_loss(dx, dy, sx, sy, rdx, rdy, rsx, rsy, t12v):
    mesh = plsc.VectorSubcoreMesh(core_axis_name="c", subcore_axis_name="s")
    cp = pltpu.CompilerParams()
    if "needs_layout_passes" in pltpu.CompilerParams.__dataclass_fields__:
        cp = dataclasses.replace(cp, needs_layout_passes=False)

    @pl.kernel(
        compiler_params=cp,
        out_type=jax.ShapeDtypeStruct((2, 16, 16), jnp.float32),
        mesh=mesh,
        scratch_types=[
            pltpu.VMEM((_M,), jnp.float32),
            pltpu.VMEM((_M,), jnp.float32),
            pltpu.VMEM((_M,), jnp.int32),
            pltpu.VMEM((_M,), jnp.int32),
            pltpu.VMEM((_M,), jnp.float32),
            pltpu.VMEM((_M,), jnp.float32),
            pltpu.VMEM((16,), jnp.float32),
            pltpu.VMEM((16,), jnp.float32),
            pltpu.VMEM((16,), jnp.float32),
        ],
    )
    def sort_loss_kernel(dx_hbm, dy_hbm, sx_hbm, sy_hbm,
                         rdx_hbm, rdy_hbm, rsx_hbm, rsy_hbm, t12_hbm,
                         out_hbm, vx, vy, rix, riy, sbx, sby, acc, vt, ov):
        cid = jax.lax.axis_index("c")
        sid = jax.lax.axis_index("s")

        def pair_loss(xv_hbm, yv_hbm, rx_hbm, ry_hbm, n):
            pltpu.sync_copy(xv_hbm, vx.at[pl.ds(0, n)])
            pltpu.sync_copy(yv_hbm, vy.at[pl.ds(0, n)])
            pltpu.sync_copy(rx_hbm, rix.at[pl.ds(0, n)])
            pltpu.sync_copy(ry_hbm, riy.at[pl.ds(0, n)])
            acc[...] = jnp.zeros((16,), jnp.float32)

            @pl.loop(0, n // 16)
            def _(c):
                off = c * 16
                plsc.store_scatter(sbx, [rix[pl.ds(off, 16)]],
                                   vx[pl.ds(off, 16)])
                plsc.store_scatter(sby, [riy[pl.ds(off, 16)]],
                                   vy[pl.ds(off, 16)])

            @pl.loop(0, n // 16)
            def _(c):
                off = c * 16
                d = sbx[pl.ds(off, 16)] - sby[pl.ds(off, 16)]
                acc[...] += d * d

            return jnp.sum(acc[...]) * (1.0 / n)

        degree_loss = pair_loss(dx_hbm, dy_hbm, rdx_hbm, rdy_hbm, _NC)
        size_loss = pair_loss(sx_hbm, sy_hbm, rsx_hbm, rsy_hbm, _M)
        pltpu.sync_copy(t12_hbm, vt)
        ov[...] = vt[...] + degree_loss + size_loss
        pltpu.sync_copy(ov, out_hbm.at[cid, sid])

    return sort_loss_kernel(dx, dy, sx, sy, rdx, rdy, rsx, rsy, t12v)


@jax.jit
def _run(theta_log, seed_prob, Ic, Fc):
    theta_t = theta_log.T
    seed2 = seed_prob.reshape(1, _NC)
    fc_f = Fc.astype(jnp.float32)
    outs = pl.pallas_call(
        _main_body,
        out_shape=[
            jax.ShapeDtypeStruct((1, 16), jnp.float32),
            jax.ShapeDtypeStruct((1, _NC), jnp.float32),
            jax.ShapeDtypeStruct((1, _NC), jnp.float32),
            jax.ShapeDtypeStruct((1, _M), jnp.float32),
            jax.ShapeDtypeStruct((1, _M), jnp.float32),
            jax.ShapeDtypeStruct((1, _NC), jnp.int32),
            jax.ShapeDtypeStruct((1, _NC), jnp.int32),
            jax.ShapeDtypeStruct((1, _M), jnp.int32),
            jax.ShapeDtypeStruct((1, _M), jnp.int32),
        ],
    )(theta_t, seed2, Ic, fc_f)
    t12, dxv, dyv, sxv, syv, rdx, rdy, rsx, rsy = outs
    out = _sc_sort_loss(dxv.reshape(_NC), dyv.reshape(_NC),
                        sxv.reshape(_M), syv.reshape(_M),
                        rdx.reshape(_NC), rdy.reshape(_NC),
                        rsx.reshape(_M), rsy.reshape(_M),
                        t12.reshape(16))
    return out[0, 0, 0]


def kernel(theta_log, seed_prob, Ic, Fc):
    return _run(theta_log, seed_prob, Ic, Fc)


# speedup vs baseline: 1.1104x; 1.1104x over previous
import dataclasses

import jax
import jax.numpy as jnp
from jax.experimental import pallas as pl
from jax.experimental.pallas import tpu as pltpu
from jax.experimental.pallas import tpu_sc as plsc

_M, _NC, _K = 1024, 512, 32
_HI = jax.lax.Precision.HIGHEST


def _rank_desc_row(y_col, y_row, n):
    gt = (y_col > y_row).astype(jnp.float32)
    k_idx = jax.lax.broadcasted_iota(jnp.int32, (n, n), 0)
    j_idx = jax.lax.broadcasted_iota(jnp.int32, (n, n), 1)
    tie = ((y_col == y_row) & (k_idx < j_idx)).astype(jnp.float32)
    return jnp.sum(gt + tie, axis=0, keepdims=True).astype(jnp.int32)


def _main_body(theta_t_ref, seed_ref, ic_ref, fc_ref,
               t12_ref, vals_ref, ranks_ref):
    theta = jnp.log(jax.nn.sigmoid(theta_t_ref[...]))
    t0 = theta[0:1, :]
    t1 = theta[1:2, :]
    t2 = theta[2:3, :]
    c0 = jnp.sum(t0)
    u = t1 - t0
    v = t0 - 2.0 * t1 + t2

    fc = fc_ref[...]
    dimn = (((1,), (1,)), ((), ()))
    s_col = jax.lax.dot_general(fc, u, dimn, precision=_HI)
    s_row = jax.lax.dot_general(u, fc, dimn, precision=_HI)
    g = jax.lax.dot_general(fc * v, fc, dimn, precision=_HI)
    p = c0 + s_col + s_row + g
    i_idx = jax.lax.broadcasted_iota(jnp.int32, (_NC, _NC), 0)
    j_idx = jax.lax.broadcasted_iota(jnp.int32, (_NC, _NC), 1)
    p = jnp.where(i_idx == j_idx, 0.0, p)
    e = jnp.exp(p)

    sp = seed_ref[...]
    sp_max = jnp.max(sp)
    es = jnp.exp(sp - sp_max)
    seed_row = es / jnp.sum(es)
    mask = (ic_ref[...] == 1).astype(jnp.float32)
    group_sum = jax.lax.dot_general(mask, seed_row, dimn, precision=_HI)
    w = mask * (seed_row / group_sum)

    s_mat = jax.lax.dot_general(w, e, (((1,), (0,)), ((), ())),
                                precision=_HI)

    log_s = jnp.log(jnp.where(mask > 0, s_mat, 1.0))
    others = jnp.log1p(-jnp.where(mask > 0, 0.0, s_mat))
    loss = -jnp.sum(log_s) - jnp.sum(others)

    ones_m = jnp.ones((1, _M), dtype=jnp.float32)
    ones_nc = jnp.ones((1, _NC), dtype=jnp.float32)
    dim_c0 = (((1,), (0,)), ((), ()))
    dim_rev = (((0,), (1,)), ((), ()))
    d_x_row = jax.lax.dot_general(ones_m, s_mat, dim_c0, precision=_HI)
    d_x_col = jax.lax.dot_general(s_mat, ones_m, dim_rev, precision=_HI)
    d_y_row = jax.lax.dot_general(ones_m, mask, dim_c0, precision=_HI)
    d_y_col = jax.lax.dot_general(mask, ones_m, dim_rev, precision=_HI)
    s_x_col = jax.lax.dot_general(s_mat, ones_nc, dimn, precision=_HI)
    s_x_row = jax.lax.dot_general(ones_nc, s_mat, dimn, precision=_HI)
    s_y_col = jax.lax.dot_general(mask, ones_nc, dimn, precision=_HI)
    s_y_row = jax.lax.dot_general(ones_nc, mask, dimn, precision=_HI)

    t12_ref[...] = jnp.full((1, 16), loss, dtype=jnp.float32)
    vals_ref[0:1, 0:_NC] = d_x_row
    vals_ref[0:1, _NC:2 * _NC] = d_y_row
    vals_ref[0:1, 2 * _NC:2 * _NC + _M] = s_x_row
    vals_ref[0:1, 2 * _NC + _M:2 * _NC + 2 * _M] = s_y_row
    ranks_ref[0:1, 0:_NC] = _rank_desc_row(d_x_col, d_x_row, _NC)
    ranks_ref[0:1, _NC:2 * _NC] = _rank_desc_row(d_y_col, d_y_row, _NC)
    ranks_ref[0:1, 2 * _NC:2 * _NC + _M] = _rank_desc_row(s_x_col, s_x_row, _M)
    ranks_ref[0:1, 2 * _NC + _M:2 * _NC + 2 * _M] = _rank_desc_row(
        s_y_col, s_y_row, _M)


def _sc_sort_loss(vals, ranks, t12v):
    mesh = plsc.VectorSubcoreMesh(core_axis_name="c", subcore_axis_name="s")
    cp = pltpu.CompilerParams()
    if "needs_layout_passes" in pltpu.CompilerParams.__dataclass_fields__:
        cp = dataclasses.replace(cp, needs_layout_passes=False)

    @pl.kernel(
        compiler_params=cp,
        out_type=jax.ShapeDtypeStruct((2, 16, 16), jnp.float32),
        mesh=mesh,
        scratch_types=[
            pltpu.VMEM((2 * _M,), jnp.float32),
            pltpu.VMEM((2 * _M,), jnp.int32),
            pltpu.VMEM((_M,), jnp.float32),
            pltpu.VMEM((_M,), jnp.float32),
            pltpu.VMEM((16,), jnp.float32),
            pltpu.VMEM((16,), jnp.float32),
            pltpu.VMEM((16,), jnp.float32),
        ],
    )
    def sort_loss_kernel(vals_hbm, ranks_hbm, t12_hbm, out_hbm,
                         vv, rr, sbx, sby, acc, vt, ov):
        cid = jax.lax.axis_index("c")
        sid = jax.lax.axis_index("s")

        def pair_loss(base, n, extra):
            pltpu.sync_copy(vals_hbm.at[0, pl.ds(base, 2 * n)],
                            vv.at[pl.ds(0, 2 * n)])
            pltpu.sync_copy(ranks_hbm.at[0, pl.ds(base, 2 * n)],
                            rr.at[pl.ds(0, 2 * n)])
            acc[...] = jnp.zeros((16,), jnp.float32)

            @pl.loop(0, n // 16)
            def _(c):
                off = c * 16
                plsc.store_scatter(sbx, [rr[pl.ds(off, 16)]],
                                   vv[pl.ds(off, 16)])
                plsc.store_scatter(sby, [rr[pl.ds(n + off, 16)]],
                                   vv[pl.ds(n + off, 16)])

            @pl.loop(0, n // 16)
            def _(c):
                off = c * 16
                d = sbx[pl.ds(off, 16)] - sby[pl.ds(off, 16)]
                acc[...] += d * d

            ov[...] = extra + jnp.sum(acc[...]) * (1.0 / n)
            pltpu.sync_copy(ov, out_hbm.at[cid, sid])

        @pl.when(jnp.logical_and(cid == 0, sid == 0))
        def _():
            pltpu.sync_copy(t12_hbm.at[0], vt)
            pair_loss(0, _NC, vt[...])

        @pl.when(jnp.logical_and(cid == 1, sid == 0))
        def _():
            pair_loss(2 * _NC, _M, jnp.zeros((16,), jnp.float32))

    return sort_loss_kernel(vals, ranks, t12v)


@jax.jit
def _run(theta_log, seed_prob, Ic, Fc):
    theta_t = theta_log.T
    seed2 = seed_prob.reshape(1, _NC)
    fc_f = Fc.astype(jnp.float32)
    nbuf = 2 * _NC + 2 * _M
    t12, vals, ranks = pl.pallas_call(
        _main_body,
        out_shape=[
            jax.ShapeDtypeStruct((1, 16), jnp.float32),
            jax.ShapeDtypeStruct((1, nbuf), jnp.float32),
            jax.ShapeDtypeStruct((1, nbuf), jnp.int32),
        ],
    )(theta_t, seed2, Ic, fc_f)
    out = _sc_sort_loss(vals, ranks, t12)
    return out[0, 0, 0] + out[1, 0, 0]


def kernel(theta_log, seed_prob, Ic, Fc):
    return _run(theta_log, seed_prob, Ic, Fc)
